# Initial kernel scaffold; baseline (speedup 1.0000x reference)
import functools
import jax
import jax.numpy as jnp
import numpy as np
from jax import lax
from jax.experimental import pallas as pl
from jax.experimental.pallas import tpu as pltpu
from jax.experimental.pallas import tpu_sc as plsc

N_ATOMS = 10000
DIM = 128
N_BASIS = 8
R_CUT = 5.0
NORM_FACTOR = 16.0

NC = 2
NS = 16
CHUNK = 128

_CENTERS = np.linspace(0.0, R_CUT, N_BASIS).astype(np.float32)


def _gather_sc(node_info, idx_j, e_pad):
    per_tile = e_pad // (NC * NS)
    n_chunk = per_tile // CHUNK
    mesh = plsc.VectorSubcoreMesh(core_axis_name="c", subcore_axis_name="s")

    @functools.partial(
        pl.kernel,
        out_type=jax.ShapeDtypeStruct((e_pad, DIM), jnp.float32),
        mesh=mesh,
        scratch_types=[
            pltpu.VMEM((CHUNK,), jnp.int32),
            pltpu.VMEM((CHUNK, DIM), jnp.float32),
            pltpu.SemaphoreType.DMA,
        ],
    )
    def gather_k(table_hbm, idx_hbm, out_hbm, idx_v, rows_v, sem):
        wid = lax.axis_index("s") * NC + lax.axis_index("c")
        base = wid * per_tile

        def body(i, _):
            off = base + i * CHUNK
            pltpu.sync_copy(idx_hbm.at[pl.ds(off, CHUNK)], idx_v)
            pltpu.async_copy(table_hbm.at[idx_v], rows_v, sem).wait()
            pltpu.sync_copy(rows_v, out_hbm.at[pl.ds(off, CHUNK)])
            return 0

        lax.fori_loop(0, n_chunk, body, 0)

    return gather_k(node_info, idx_j)


def _edge_tc(rij, xj, Wr1, br1, Wr2, br2, e_pad):
    BE = 2048
    grid = e_pad // BE

    def body(rij_ref, xj_ref, wr1_ref, br1_ref, wr2_ref, br2_ref, out_ref):
        r = rij_ref[...]
        d = jnp.sqrt(jnp.sum(r * r, axis=1, keepdims=True))
        unit = r / (d + 1e-9)
        centers = jnp.asarray(_CENTERS)[None, :]
        rbf = jnp.exp(-((d - centers) ** 2) / 0.5)
        cut = 0.5 * (jnp.cos(jnp.pi * jnp.clip(d, 0.0, R_CUT) / R_CUT) + 1.0)
        h = jnp.dot(rbf, wr1_ref[...], preferred_element_type=jnp.float32)
        h = h + br1_ref[...]
        h = h * jax.nn.sigmoid(h)
        filt = jnp.dot(h, wr2_ref[...], preferred_element_type=jnp.float32)
        filt = (filt + br2_ref[...]) * cut
        xj = xj_ref[...]
        m0 = filt[:, :DIM] * xj
        w1 = filt[:, DIM:] * xj
        out_ref[0] = m0
        out_ref[1] = w1 * unit[:, 0:1]
        out_ref[2] = w1 * unit[:, 1:2]
        out_ref[3] = w1 * unit[:, 2:3]

    return pl.pallas_call(
        body,
        grid=(grid,),
        in_specs=[
            pl.BlockSpec((BE, 3), lambda i: (i, 0)),
            pl.BlockSpec((BE, DIM), lambda i: (i, 0)),
            pl.BlockSpec((N_BASIS, 64), lambda i: (0, 0)),
            pl.BlockSpec((1, 64), lambda i: (0, 0)),
            pl.BlockSpec((64, 2 * DIM), lambda i: (0, 0)),
            pl.BlockSpec((1, 2 * DIM), lambda i: (0, 0)),
        ],
        out_specs=pl.BlockSpec((4, BE, DIM), lambda i: (0, i, 0)),
        out_shape=jax.ShapeDtypeStruct((4, e_pad, DIM), jnp.float32),
    )(rij, xj, Wr1, br1.reshape(1, 64), Wr2, br2.reshape(1, 2 * DIM))


def _scatter_sc(planes, idx_i, zeros, e_pad):
    per_tile = e_pad // NS
    n_chunk = per_tile // CHUNK
    rows_per_tile = N_ATOMS // NS
    mesh = plsc.VectorSubcoreMesh(core_axis_name="c", subcore_axis_name="s")

    @functools.partial(
        pl.kernel,
        out_type=jax.ShapeDtypeStruct((4, N_ATOMS, DIM), jnp.float32),
        mesh=mesh,
        scratch_types=[
            pltpu.VMEM((CHUNK,), jnp.int32),
            pltpu.VMEM((CHUNK, DIM), jnp.float32),
            pltpu.VMEM_SHARED((N_ATOMS, DIM), jnp.float32),
        ],
    )
    def scatter_k(planes_hbm, idx_hbm, zeros_hbm, out_hbm, idx_v, rows_v, acc):
        cid = lax.axis_index("c")
        sid = lax.axis_index("s")
        nbase = sid * rows_per_tile

        def do_plane(p):
            pltpu.sync_copy(zeros_hbm.at[pl.ds(nbase, rows_per_tile)],
                            acc.at[pl.ds(nbase, rows_per_tile)])
            plsc.subcore_barrier()

            def body(i, _):
                off = (sid * n_chunk + i) * CHUNK
                pltpu.sync_copy(idx_hbm.at[pl.ds(off, CHUNK)], idx_v)
                pltpu.sync_copy(planes_hbm.at[p].at[pl.ds(off, CHUNK)], rows_v)
                pltpu.sync_copy(rows_v, acc.at[idx_v], add=True)
                return 0

            lax.fori_loop(0, n_chunk, body, 0)
            plsc.subcore_barrier()
            pltpu.sync_copy(acc.at[pl.ds(nbase, rows_per_tile)],
                            out_hbm.at[p].at[pl.ds(nbase, rows_per_tile)])
            plsc.subcore_barrier()

        for r in range(2):
            for c in range(NC):
                @pl.when(cid == c)
                def _():
                    do_plane(2 * c + r)

    return scatter_k(planes, idx_i, zeros)


def _node_tc(aggs, node_info, W0, b0, W1, gw, gb):
    BN = 2500
    grid = N_ATOMS // BN
    inv = 1.0 / NORM_FACTOR

    def body(agg_ref, node_ref, w0_ref, b0_ref, w1_ref, gw_ref, gb_ref,
             out0_ref, out1_ref):
        a0 = agg_ref[0] * inv
        s0 = jnp.dot(a0, w0_ref[...], preferred_element_type=jnp.float32)
        s0 = s0 + b0_ref[...]
        out0_ref[...] = node_ref[...] + s0 * jax.nn.sigmoid(s0)
        w1 = w1_ref[...]
        s1x = jnp.dot(agg_ref[1] * inv, w1, preferred_element_type=jnp.float32)
        s1y = jnp.dot(agg_ref[2] * inv, w1, preferred_element_type=jnp.float32)
        s1z = jnp.dot(agg_ref[3] * inv, w1, preferred_element_type=jnp.float32)
        norm = jnp.sqrt(s1x * s1x + s1y * s1y + s1z * s1z + 1e-9)
        gate = jax.nn.sigmoid(norm * gw_ref[...] + gb_ref[...])
        out1_ref[0] = s1x * gate
        out1_ref[1] = s1y * gate
        out1_ref[2] = s1z * gate

    return pl.pallas_call(
        body,
        grid=(grid,),
        in_specs=[
            pl.BlockSpec((4, BN, DIM), lambda i: (0, i, 0)),
            pl.BlockSpec((BN, DIM), lambda i: (i, 0)),
            pl.BlockSpec((DIM, DIM), lambda i: (0, 0)),
            pl.BlockSpec((1, DIM), lambda i: (0, 0)),
            pl.BlockSpec((DIM, DIM), lambda i: (0, 0)),
            pl.BlockSpec((1, DIM), lambda i: (0, 0)),
            pl.BlockSpec((1, DIM), lambda i: (0, 0)),
        ],
        out_specs=[
            pl.BlockSpec((BN, DIM), lambda i: (i, 0)),
            pl.BlockSpec((3, BN, DIM), lambda i: (0, i, 0)),
        ],
        out_shape=[
            jax.ShapeDtypeStruct((N_ATOMS, DIM), jnp.float32),
            jax.ShapeDtypeStruct((3, N_ATOMS, DIM), jnp.float32),
        ],
    )(aggs, node_info, W0, b0.reshape(1, DIM), W1,
      gw.reshape(1, DIM), gb.reshape(1, DIM))


def kernel(node_info_way0, rij, Wr1, br1, Wr2, br2, W0, b0, W1, gw, gb,
           idx_i, idx_j, atomic_number):
    n_edges = rij.shape[0]
    unit_e = NC * NS * CHUNK
    e_pad = ((n_edges + unit_e - 1) // unit_e) * unit_e
    pad = e_pad - n_edges
    rij_p = jnp.concatenate(
        [rij, jnp.broadcast_to(jnp.array([2 * R_CUT, 0.0, 0.0], rij.dtype),
                               (pad, 3))], axis=0)
    idx_i_p = jnp.concatenate([idx_i, jnp.zeros((pad,), jnp.int32)])
    idx_j_p = jnp.concatenate([idx_j, jnp.zeros((pad,), jnp.int32)])

    xj = _gather_sc(node_info_way0, idx_j_p, e_pad)
    planes = _edge_tc(rij_p, xj, Wr1, br1, Wr2, br2, e_pad)
    zeros = jnp.zeros((N_ATOMS, DIM), jnp.float32)
    aggs = _scatter_sc(planes, idx_i_p, zeros, e_pad)
    out0, o1 = _node_tc(aggs, node_info_way0, W0, b0, W1, gw, gb)
    out1 = jnp.moveaxis(o1, 0, -1)
    return (out0, out1)


# speedup vs baseline: 15.1169x; 15.1169x over previous
import functools
import jax
import jax.numpy as jnp
import numpy as np
from jax import lax
from jax.experimental import pallas as pl
from jax.experimental.pallas import tpu as pltpu
from jax.experimental.pallas import tpu_sc as plsc

N_ATOMS = 10000
DIM = 128
N_BASIS = 8
R_CUT = 5.0
NORM_FACTOR = 16.0

NC = 2
NS = 16
CHUNK = 128

_CENTERS = np.linspace(0.0, R_CUT, N_BASIS).astype(np.float32)


def _gather_sc(node_info, idx_j, e_pad):
    per_tile = e_pad // (NC * NS)
    n_chunk = per_tile // CHUNK
    mesh = plsc.VectorSubcoreMesh(core_axis_name="c", subcore_axis_name="s")

    @functools.partial(
        pl.kernel,
        out_type=jax.ShapeDtypeStruct((e_pad, DIM), jnp.float32),
        mesh=mesh,
        scratch_types=[
            pltpu.VMEM((CHUNK,), jnp.int32),
            pltpu.VMEM((CHUNK, DIM), jnp.float32),
            pltpu.SemaphoreType.DMA,
        ],
    )
    def gather_k(table_hbm, idx_hbm, out_hbm, idx_v, rows_v, sem):
        wid = lax.axis_index("s") * NC + lax.axis_index("c")
        base = wid * per_tile

        def body(i, _):
            off = base + i * CHUNK
            pltpu.sync_copy(idx_hbm.at[pl.ds(off, CHUNK)], idx_v)
            pltpu.async_copy(table_hbm.at[idx_v], rows_v, sem).wait()
            pltpu.sync_copy(rows_v, out_hbm.at[pl.ds(off, CHUNK)])
            return 0

        lax.fori_loop(0, n_chunk, body, 0)

    return gather_k(node_info, idx_j)


def _edge_tc(rij, xj, Wr1, br1, Wr2, br2, e_pad):
    BE = 2048
    grid = e_pad // BE

    def body(rij_ref, xj_ref, wr1_ref, br1_ref, wr2_ref, br2_ref, out_ref):
        r = rij_ref[...]
        d = jnp.sqrt(jnp.sum(r * r, axis=1, keepdims=True))
        unit = r / (d + 1e-9)
        centers = lax.broadcasted_iota(jnp.int32, (1, N_BASIS), 1).astype(
            jnp.float32) * (R_CUT / (N_BASIS - 1))
        rbf = jnp.exp(-((d - centers) ** 2) / 0.5)
        cut = 0.5 * (jnp.cos(jnp.pi * jnp.clip(d, 0.0, R_CUT) / R_CUT) + 1.0)
        h = jnp.dot(rbf, wr1_ref[...], preferred_element_type=jnp.float32)
        h = h + br1_ref[...]
        h = h * jax.nn.sigmoid(h)
        filt = jnp.dot(h, wr2_ref[...], preferred_element_type=jnp.float32)
        filt = (filt + br2_ref[...]) * cut
        xj = xj_ref[...]
        m0 = filt[:, :DIM] * xj
        w1 = filt[:, DIM:] * xj
        out_ref[0] = m0
        out_ref[1] = w1 * unit[:, 0:1]
        out_ref[2] = w1 * unit[:, 1:2]
        out_ref[3] = w1 * unit[:, 2:3]

    return pl.pallas_call(
        body,
        grid=(grid,),
        in_specs=[
            pl.BlockSpec((BE, 3), lambda i: (i, 0)),
            pl.BlockSpec((BE, DIM), lambda i: (i, 0)),
            pl.BlockSpec((N_BASIS, 64), lambda i: (0, 0)),
            pl.BlockSpec((1, 64), lambda i: (0, 0)),
            pl.BlockSpec((64, 2 * DIM), lambda i: (0, 0)),
            pl.BlockSpec((1, 2 * DIM), lambda i: (0, 0)),
        ],
        out_specs=pl.BlockSpec((4, BE, DIM), lambda i: (0, i, 0)),
        out_shape=jax.ShapeDtypeStruct((4, e_pad, DIM), jnp.float32),
    )(rij, xj, Wr1, br1.reshape(1, 64), Wr2, br2.reshape(1, 2 * DIM))


def _scatter_sc(planes, idx_i, zeros, e_pad, n_pad):
    per_tile = e_pad // NS
    n_chunk = per_tile // CHUNK
    rows_per_tile = n_pad // NS
    mesh = plsc.VectorSubcoreMesh(core_axis_name="c", subcore_axis_name="s")

    @functools.partial(
        pl.kernel,
        out_type=jax.ShapeDtypeStruct((4, n_pad, DIM), jnp.float32),
        mesh=mesh,
        scratch_types=[
            pltpu.VMEM((CHUNK,), jnp.int32),
            pltpu.VMEM((CHUNK, DIM), jnp.float32),
            pltpu.VMEM_SHARED((n_pad, DIM), jnp.float32),
        ],
    )
    def scatter_k(planes_hbm, idx_hbm, zeros_hbm, out_hbm, idx_v, rows_v, acc):
        cid = lax.axis_index("c")
        sid = lax.axis_index("s")
        nbase = sid * rows_per_tile

        def do_plane(p):
            pltpu.sync_copy(zeros_hbm.at[pl.ds(nbase, rows_per_tile)],
                            acc.at[pl.ds(nbase, rows_per_tile)])
            plsc.subcore_barrier()

            def body(i, _):
                off = (sid * n_chunk + i) * CHUNK
                pltpu.sync_copy(idx_hbm.at[pl.ds(off, CHUNK)], idx_v)
                pltpu.sync_copy(planes_hbm.at[p].at[pl.ds(off, CHUNK)], rows_v)
                pltpu.sync_copy(rows_v, acc.at[idx_v], add=True)
                return 0

            lax.fori_loop(0, n_chunk, body, 0)
            plsc.subcore_barrier()
            pltpu.sync_copy(acc.at[pl.ds(nbase, rows_per_tile)],
                            out_hbm.at[p].at[pl.ds(nbase, rows_per_tile)])
            plsc.subcore_barrier()

        for r in range(2):
            for c in range(NC):
                @pl.when(cid == c)
                def _():
                    do_plane(2 * c + r)

    return scatter_k(planes, idx_i, zeros)


def _node_tc(aggs, node_info, W0, b0, W1, gw, gb):
    BN = 2000
    grid = N_ATOMS // BN
    inv = 1.0 / NORM_FACTOR

    def body(agg_ref, node_ref, w0_ref, b0_ref, w1_ref, gw_ref, gb_ref,
             out0_ref, out1_ref):
        a0 = agg_ref[0] * inv
        s0 = jnp.dot(a0, w0_ref[...], preferred_element_type=jnp.float32)
        s0 = s0 + b0_ref[...]
        out0_ref[...] = node_ref[...] + s0 * jax.nn.sigmoid(s0)
        w1 = w1_ref[...]
        s1x = jnp.dot(agg_ref[1] * inv, w1, preferred_element_type=jnp.float32)
        s1y = jnp.dot(agg_ref[2] * inv, w1, preferred_element_type=jnp.float32)
        s1z = jnp.dot(agg_ref[3] * inv, w1, preferred_element_type=jnp.float32)
        norm = jnp.sqrt(s1x * s1x + s1y * s1y + s1z * s1z + 1e-9)
        gate = jax.nn.sigmoid(norm * gw_ref[...] + gb_ref[...])
        out1_ref[0] = s1x * gate
        out1_ref[1] = s1y * gate
        out1_ref[2] = s1z * gate

    return pl.pallas_call(
        body,
        grid=(grid,),
        in_specs=[
            pl.BlockSpec((4, BN, DIM), lambda i: (0, i, 0)),
            pl.BlockSpec((BN, DIM), lambda i: (i, 0)),
            pl.BlockSpec((DIM, DIM), lambda i: (0, 0)),
            pl.BlockSpec((1, DIM), lambda i: (0, 0)),
            pl.BlockSpec((DIM, DIM), lambda i: (0, 0)),
            pl.BlockSpec((1, DIM), lambda i: (0, 0)),
            pl.BlockSpec((1, DIM), lambda i: (0, 0)),
        ],
        out_specs=[
            pl.BlockSpec((BN, DIM), lambda i: (i, 0)),
            pl.BlockSpec((3, BN, DIM), lambda i: (0, i, 0)),
        ],
        out_shape=[
            jax.ShapeDtypeStruct((N_ATOMS, DIM), jnp.float32),
            jax.ShapeDtypeStruct((3, N_ATOMS, DIM), jnp.float32),
        ],
    )(aggs, node_info, W0, b0.reshape(1, DIM), W1,
      gw.reshape(1, DIM), gb.reshape(1, DIM))


def kernel(node_info_way0, rij, Wr1, br1, Wr2, br2, W0, b0, W1, gw, gb,
           idx_i, idx_j, atomic_number):
    n_edges = rij.shape[0]
    unit_e = NC * NS * CHUNK
    e_pad = ((n_edges + unit_e - 1) // unit_e) * unit_e
    pad = e_pad - n_edges
    rij_p = jnp.concatenate(
        [rij, jnp.broadcast_to(jnp.array([2 * R_CUT, 0.0, 0.0], rij.dtype),
                               (pad, 3))], axis=0)
    idx_i_p = jnp.concatenate([idx_i, jnp.zeros((pad,), jnp.int32)])
    idx_j_p = jnp.concatenate([idx_j, jnp.zeros((pad,), jnp.int32)])

    xj = _gather_sc(node_info_way0, idx_j_p, e_pad)
    planes = _edge_tc(rij_p, xj, Wr1, br1, Wr2, br2, e_pad)
    n_pad = ((N_ATOMS + NS * 8 - 1) // (NS * 8)) * (NS * 8)
    zeros = jnp.zeros((n_pad, DIM), jnp.float32)
    aggs = _scatter_sc(planes, idx_i_p, zeros, e_pad, n_pad)[:, :N_ATOMS]
    out0, o1 = _node_tc(aggs, node_info_way0, W0, b0, W1, gw, gb)
    out1 = jnp.moveaxis(o1, 0, -1)
    return (out0, out1)


# speedup vs baseline: 19.4786x; 1.2885x over previous
import functools
import jax
import jax.numpy as jnp
import numpy as np
from jax import lax
from jax.experimental import pallas as pl
from jax.experimental.pallas import tpu as pltpu
from jax.experimental.pallas import tpu_sc as plsc

N_ATOMS = 10000
DIM = 128
N_BASIS = 8
R_CUT = 5.0
NORM_FACTOR = 16.0

NC = 2
NS = 16
CHUNK = 128

_CENTERS = np.linspace(0.0, R_CUT, N_BASIS).astype(np.float32)


def _gather_sc(node_info, idx_j2d, e_pad):
    per_tile = e_pad // (NC * NS)
    n_chunk = per_tile // CHUNK
    n_pair = n_chunk // 2
    mesh = plsc.VectorSubcoreMesh(core_axis_name="c", subcore_axis_name="s")

    @functools.partial(
        pl.kernel,
        out_type=jax.ShapeDtypeStruct((e_pad, DIM), jnp.float32),
        mesh=mesh,
        scratch_types=[
            pltpu.VMEM((n_chunk, CHUNK), jnp.int32),
            pltpu.VMEM((CHUNK, DIM), jnp.float32),
            pltpu.VMEM((CHUNK, DIM), jnp.float32),
            pltpu.SemaphoreType.DMA,
            pltpu.SemaphoreType.DMA,
        ],
    )
    def gather_k(table_hbm, idx_hbm, out_hbm, idx2d, buf0, buf1, sem0, sem1):
        wid = lax.axis_index("s") * NC + lax.axis_index("c")
        base = wid * per_tile
        cbase = wid * n_chunk
        pltpu.sync_copy(idx_hbm.at[pl.ds(cbase, n_chunk)], idx2d)
        pltpu.async_copy(table_hbm.at[idx2d.at[0]], buf0, sem0)

        def body(g, _):
            i0 = 2 * g
            pltpu.async_copy(table_hbm.at[idx2d.at[i0 + 1]], buf1, sem1)
            pltpu.make_async_copy(table_hbm.at[idx2d.at[i0]], buf0, sem0).wait()
            pltpu.sync_copy(buf0, out_hbm.at[pl.ds(base + i0 * CHUNK, CHUNK)])

            @pl.when(g < n_pair - 1)
            def _():
                pltpu.async_copy(table_hbm.at[idx2d.at[i0 + 2]], buf0, sem0)

            pltpu.make_async_copy(table_hbm.at[idx2d.at[i0 + 1]], buf1,
                                  sem1).wait()
            pltpu.sync_copy(buf1,
                            out_hbm.at[pl.ds(base + (i0 + 1) * CHUNK, CHUNK)])
            return 0

        lax.fori_loop(0, n_pair, body, 0)

    return gather_k(node_info, idx_j2d)


def _edge_tc(rij, xj, Wr1, br1, Wr2, br2, e_pad):
    BE = 2048
    grid = e_pad // BE

    def body(rij_ref, xj_ref, wr1_ref, br1_ref, wr2_ref, br2_ref, out_ref):
        r = rij_ref[...]
        d = jnp.sqrt(jnp.sum(r * r, axis=1, keepdims=True))
        unit = r / (d + 1e-9)
        centers = lax.broadcasted_iota(jnp.int32, (1, N_BASIS), 1).astype(
            jnp.float32) * (R_CUT / (N_BASIS - 1))
        rbf = jnp.exp(-((d - centers) ** 2) / 0.5)
        cut = 0.5 * (jnp.cos(jnp.pi * jnp.clip(d, 0.0, R_CUT) / R_CUT) + 1.0)
        h = jnp.dot(rbf, wr1_ref[...], preferred_element_type=jnp.float32)
        h = h + br1_ref[...]
        h = h * jax.nn.sigmoid(h)
        filt = jnp.dot(h, wr2_ref[...], preferred_element_type=jnp.float32)
        filt = (filt + br2_ref[...]) * cut
        xj = xj_ref[...]
        m0 = filt[:, :DIM] * xj
        w1 = filt[:, DIM:] * xj
        out_ref[0] = m0
        out_ref[1] = w1 * unit[:, 0:1]
        out_ref[2] = w1 * unit[:, 1:2]
        out_ref[3] = w1 * unit[:, 2:3]

    return pl.pallas_call(
        body,
        grid=(grid,),
        in_specs=[
            pl.BlockSpec((BE, 3), lambda i: (i, 0)),
            pl.BlockSpec((BE, DIM), lambda i: (i, 0)),
            pl.BlockSpec((N_BASIS, 64), lambda i: (0, 0)),
            pl.BlockSpec((1, 64), lambda i: (0, 0)),
            pl.BlockSpec((64, 2 * DIM), lambda i: (0, 0)),
            pl.BlockSpec((1, 2 * DIM), lambda i: (0, 0)),
        ],
        out_specs=pl.BlockSpec((4, BE, DIM), lambda i: (0, i, 0)),
        out_shape=jax.ShapeDtypeStruct((4, e_pad, DIM), jnp.float32),
    )(rij, xj, Wr1, br1.reshape(1, 64), Wr2, br2.reshape(1, 2 * DIM))


def _scatter_sc(planes, idx_i, zeros, e_pad, n_pad):
    per_tile = e_pad // NS
    n_chunk = per_tile // CHUNK
    rows_per_tile = n_pad // NS
    mesh = plsc.VectorSubcoreMesh(core_axis_name="c", subcore_axis_name="s")

    @functools.partial(
        pl.kernel,
        out_type=jax.ShapeDtypeStruct((4, n_pad, DIM), jnp.float32),
        mesh=mesh,
        scratch_types=[
            pltpu.VMEM((n_chunk, CHUNK), jnp.int32),
            pltpu.VMEM((CHUNK, DIM), jnp.float32),
            pltpu.VMEM((CHUNK, DIM), jnp.float32),
            pltpu.VMEM_SHARED((n_pad, DIM), jnp.float32),
            pltpu.SemaphoreType.DMA,
            pltpu.SemaphoreType.DMA,
        ],
    )
    def scatter_k(planes_hbm, idx_hbm, zeros_hbm, out_hbm, idx2d, buf0, buf1,
                  acc, sem0, sem1):
        cid = lax.axis_index("c")
        sid = lax.axis_index("s")
        nbase = sid * rows_per_tile
        ebase = sid * per_tile
        n_pair = n_chunk // 2

        pltpu.sync_copy(idx_hbm.at[pl.ds(sid * n_chunk, n_chunk)], idx2d)

        def do_plane(p):
            pltpu.sync_copy(zeros_hbm.at[pl.ds(nbase, rows_per_tile)],
                            acc.at[pl.ds(nbase, rows_per_tile)])
            plsc.subcore_barrier()
            plane = planes_hbm.at[p]
            pltpu.async_copy(plane.at[pl.ds(ebase, CHUNK)], buf0, sem0)

            def body(q, _):
                o0 = ebase + 2 * q * CHUNK
                pltpu.async_copy(plane.at[pl.ds(o0 + CHUNK, CHUNK)], buf1,
                                 sem1)
                pltpu.make_async_copy(plane.at[pl.ds(o0, CHUNK)], buf0,
                                      sem0).wait()
                pltpu.sync_copy(buf0, acc.at[idx2d.at[2 * q]], add=True)

                @pl.when(q < n_pair - 1)
                def _():
                    pltpu.async_copy(plane.at[pl.ds(o0 + 2 * CHUNK, CHUNK)],
                                     buf0, sem0)

                pltpu.make_async_copy(plane.at[pl.ds(o0 + CHUNK, CHUNK)],
                                      buf1, sem1).wait()
                pltpu.sync_copy(buf1, acc.at[idx2d.at[2 * q + 1]], add=True)
                return 0

            lax.fori_loop(0, n_pair, body, 0)
            plsc.subcore_barrier()
            pltpu.sync_copy(acc.at[pl.ds(nbase, rows_per_tile)],
                            out_hbm.at[p].at[pl.ds(nbase, rows_per_tile)])
            plsc.subcore_barrier()

        for r in range(2):
            for c in range(NC):
                @pl.when(cid == c)
                def _():
                    do_plane(2 * c + r)

    return scatter_k(planes, idx_i, zeros)


def _node_tc(aggs, node_info, W0, b0, W1, gw, gb):
    BN = 2000
    grid = N_ATOMS // BN
    inv = 1.0 / NORM_FACTOR

    def body(agg_ref, node_ref, w0_ref, b0_ref, w1_ref, gw_ref, gb_ref,
             out0_ref, out1_ref):
        a0 = agg_ref[0] * inv
        s0 = jnp.dot(a0, w0_ref[...], preferred_element_type=jnp.float32)
        s0 = s0 + b0_ref[...]
        out0_ref[...] = node_ref[...] + s0 * jax.nn.sigmoid(s0)
        w1 = w1_ref[...]
        s1x = jnp.dot(agg_ref[1] * inv, w1, preferred_element_type=jnp.float32)
        s1y = jnp.dot(agg_ref[2] * inv, w1, preferred_element_type=jnp.float32)
        s1z = jnp.dot(agg_ref[3] * inv, w1, preferred_element_type=jnp.float32)
        norm = jnp.sqrt(s1x * s1x + s1y * s1y + s1z * s1z + 1e-9)
        gate = jax.nn.sigmoid(norm * gw_ref[...] + gb_ref[...])
        out1_ref[0] = s1x * gate
        out1_ref[1] = s1y * gate
        out1_ref[2] = s1z * gate

    return pl.pallas_call(
        body,
        grid=(grid,),
        in_specs=[
            pl.BlockSpec((4, BN, DIM), lambda i: (0, i, 0)),
            pl.BlockSpec((BN, DIM), lambda i: (i, 0)),
            pl.BlockSpec((DIM, DIM), lambda i: (0, 0)),
            pl.BlockSpec((1, DIM), lambda i: (0, 0)),
            pl.BlockSpec((DIM, DIM), lambda i: (0, 0)),
            pl.BlockSpec((1, DIM), lambda i: (0, 0)),
            pl.BlockSpec((1, DIM), lambda i: (0, 0)),
        ],
        out_specs=[
            pl.BlockSpec((BN, DIM), lambda i: (i, 0)),
            pl.BlockSpec((3, BN, DIM), lambda i: (0, i, 0)),
        ],
        out_shape=[
            jax.ShapeDtypeStruct((N_ATOMS, DIM), jnp.float32),
            jax.ShapeDtypeStruct((3, N_ATOMS, DIM), jnp.float32),
        ],
    )(aggs, node_info, W0, b0.reshape(1, DIM), W1,
      gw.reshape(1, DIM), gb.reshape(1, DIM))


def kernel(node_info_way0, rij, Wr1, br1, Wr2, br2, W0, b0, W1, gw, gb,
           idx_i, idx_j, atomic_number):
    n_edges = rij.shape[0]
    unit_e = NC * NS * CHUNK
    e_pad = ((n_edges + unit_e - 1) // unit_e) * unit_e
    pad = e_pad - n_edges
    rij_p = jnp.concatenate(
        [rij, jnp.broadcast_to(jnp.array([2 * R_CUT, 0.0, 0.0], rij.dtype),
                               (pad, 3))], axis=0)
    idx_i_p = jnp.concatenate([idx_i, jnp.zeros((pad,), jnp.int32)])
    idx_j_p = jnp.concatenate([idx_j, jnp.zeros((pad,), jnp.int32)])
    idx_i2d = idx_i_p.reshape(e_pad // CHUNK, CHUNK)
    idx_j2d = idx_j_p.reshape(e_pad // CHUNK, CHUNK)

    xj = _gather_sc(node_info_way0, idx_j2d, e_pad)
    planes = _edge_tc(rij_p, xj, Wr1, br1, Wr2, br2, e_pad)
    n_pad = ((N_ATOMS + NS * 8 - 1) // (NS * 8)) * (NS * 8)
    zeros = jnp.zeros((n_pad, DIM), jnp.float32)
    aggs = _scatter_sc(planes, idx_i2d, zeros, e_pad, n_pad)
    out0, o1 = _node_tc(aggs, node_info_way0, W0, b0, W1, gw, gb)
    out1 = jnp.moveaxis(o1, 0, -1)
    return (out0, out1)
